# baseline (speedup 1.0000x reference)
import jax
import jax.numpy as jnp
from jax import lax
from jax.experimental import pallas as pl
from jax.experimental.pallas import tpu as pltpu
from jax.experimental.pallas import tpu_sc as plsc

_C, _G, _S, _L = 16, 4096, 8, 2
_B = 32
_STEPS = 3
_GAMMA = 0.01
_IG = 100.0
_IG2 = 144.26950408889634
_LN2 = 0.6931471805599453
_C1 = _GAMMA * _LN2
_C2 = 2.0 * _GAMMA

_NC, _NS = 2, 16
_NW = _NC * _NS
_SLOTS = _C * _G
_SPW = _SLOTS // _NW
_CS = 8
_RPC = _CS * _S * _L
_CPW = _SPW // _CS
_OSL = 256
_CPO = _OSL // _CS
_OBW = _SPW // _OSL
_NROWS = _SLOTS * _S * _L // _RPC


def _p1(f, xs, *cs):
    return [f(x, *cs) for x in xs]


def _p2(f, xs, ys):
    return [f(x, y) for x, y in zip(xs, ys)]


def _ptree(f, pairs_list):
    while len(pairs_list) > 1:
        nxt = [_p2(f, pairs_list[i], pairs_list[i + 1])
               for i in range(0, len(pairs_list) - 1, 2)]
        if len(pairs_list) % 2:
            nxt.append(pairs_list[-1])
        pairs_list = nxt
    return pairs_list[0]


def _gamma_log_pair(accs):
    bits = _p1(lambda a: lax.bitcast_convert_type(a, jnp.int32), accs)
    e = _p1(lambda b: ((b >> 23) & 0xFF) - 127, bits)
    man = _p1(lambda b: lax.bitcast_convert_type(
        (b & 0x007FFFFF) | 0x3F800000, jnp.float32), bits)
    t = _p1(lambda mn: mn - 1.0, man)
    s = _p2(lambda tt, d: tt / d, t, _p1(lambda tt: tt + 2.0, t))
    s2 = _p2(lambda a, b: a * b, s, s)
    p = _p1(lambda q: jnp.float32(0.2) + q * jnp.float32(1.0 / 7.0), s2)
    p = _p2(lambda q, pp: jnp.float32(1.0 / 3.0) + q * pp, s2, p)
    p = _p2(lambda q, pp: 1.0 + q * pp, s2, p)
    ef = _p1(lambda ee: ee.astype(jnp.float32) * jnp.float32(_C1), e)
    sp = _p2(lambda a, b: jnp.float32(_C2) * a * b, s, p)
    return _p2(lambda a, b: a + b, ef, sp)


def _compute_chunk(rows_v, out_v, col):
    for k in range(_CS):
        base = k * 16
        rs = [[rows_v[base + j, pl.ds(lo, 16)] for lo in (0, 16)]
              for j in range(16)]
        bs = [_p2(lambda a, b: a * b, rs[2 * s], rs[2 * s + 1])
              for s in range(_S)]
        m = _ptree(jnp.maximum, bs)
        es = [_p2(lambda b, mm: jnp.exp((b - mm) * _IG), b, m) for b in bs]
        acc = _ptree(lambda a, b: a + b, es)
        gln = _gamma_log_pair(acc)
        lse = _p2(lambda mm, l_: mm + l_, m, gln)
        out_v[pl.ds((col + k) * _B, 16)] = lse[0]
        out_v[pl.ds((col + k) * _B + 16, 16)] = lse[1]


def _stage_a_body(idx_hbm, xt_hbm, p_hbm,
                  idx_v, rows_a, rows_b, rows_c, rows_d, out_v,
                  sem_a, sem_b, sem_c, sem_d):
    cid = lax.axis_index("c")
    sid = lax.axis_index("s")
    w = sid * _NC + cid
    cc = w // 2
    gb = (w % 2) * (_G // 2)

    pltpu.sync_copy(idx_hbm.at[pl.ds(w * _CPW, _CPW), :], idx_v)

    def issue(ch, rows, sem):
        pltpu.async_copy(xt_hbm.at[idx_v.at[ch]], rows, sem)

    def wait(rows, sem):
        pltpu.make_async_copy(xt_hbm.at[idx_v.at[0]], rows, sem).wait()

    nq = _CPO // 4

    def ob_body(ob, carry):
        c0 = ob * _CPO
        issue(c0, rows_a, sem_a)
        issue(c0 + 1, rows_b, sem_b)

        def quad_body(q, c_):
            j0 = c0 + q * 4
            issue(j0 + 2, rows_c, sem_c)
            issue(j0 + 3, rows_d, sem_d)
            wait(rows_a, sem_a)
            _compute_chunk(rows_a, out_v, (q * 4) * _CS)
            wait(rows_b, sem_b)
            _compute_chunk(rows_b, out_v, (q * 4 + 1) * _CS)

            @pl.when(q < nq - 1)
            def _():
                issue(j0 + 4, rows_a, sem_a)
                issue(j0 + 5, rows_b, sem_b)

            wait(rows_c, sem_c)
            _compute_chunk(rows_c, out_v, (q * 4 + 2) * _CS)
            wait(rows_d, sem_d)
            _compute_chunk(rows_d, out_v, (q * 4 + 3) * _CS)
            return c_

        lax.fori_loop(0, nq, quad_body, 0)
        pltpu.sync_copy(
            out_v,
            p_hbm.at[pl.ds(((cc * _G + gb) + ob * _OSL) * _B, _OSL * _B)])
        return carry

    lax.fori_loop(0, _OBW, ob_body, 0)


_stage_a = pl.kernel(
    _stage_a_body,
    out_type=jax.ShapeDtypeStruct((_C * _G * _B,), jnp.float32),
    mesh=plsc.VectorSubcoreMesh(core_axis_name="c", subcore_axis_name="s"),
    compiler_params=pltpu.CompilerParams(use_tc_tiling_on_sc=False),
    scratch_types=(
        pltpu.VMEM((_CPW, _RPC), jnp.int32),
        pltpu.VMEM((_RPC, _B), jnp.float32),
        pltpu.VMEM((_RPC, _B), jnp.float32),
        pltpu.VMEM((_RPC, _B), jnp.float32),
        pltpu.VMEM((_RPC, _B), jnp.float32),
        pltpu.VMEM((_OSL * _B,), jnp.float32),
        pltpu.SemaphoreType.DMA,
        pltpu.SemaphoreType.DMA,
        pltpu.SemaphoreType.DMA,
        pltpu.SemaphoreType.DMA,
    ),
)


def _stage_b_body(p_ref, rt_ref, rnt_ref):
    m1 = jnp.max(p_ref[...])
    s1 = jnp.where(m1 > 1.0, 1.0 / m1, 1.0)
    cv = p_ref[...] * s1
    mxc = jnp.max(cv, axis=0)
    acc = jnp.sum(jnp.exp((cv - mxc[None, :, :]) * _IG), axis=0)
    lse_c = mxc + _GAMMA * jnp.log(acc)
    m2 = jnp.max(lse_c)
    rr = lse_c * jnp.where(m2 > 1.0, 1.0 / m2, 1.0)
    rc = rt_ref[...]
    mx2 = jnp.maximum(rc, rr)
    z = mx2 + _GAMMA * jnp.log(jnp.exp((rc - mx2) * _IG)
                               + jnp.exp((rr - mx2) * _IG))
    m3 = jnp.max(z)
    rnt_ref[...] = z * jnp.where(m3 > 1.0, 1.0 / m3, 1.0)


_GB = _G * _B
_ROWS128 = _GB // 128

_stage_b = pl.pallas_call(
    _stage_b_body,
    out_shape=jax.ShapeDtypeStruct((_ROWS128, 128), jnp.float32),
)


def _tr_body(rt_ref, r_ref):
    r_ref[...] = rt_ref[...].T


_tr = pl.pallas_call(
    _tr_body,
    out_shape=jax.ShapeDtypeStruct((_B, _G), jnp.float32),
)


def kernel(x, I):
    idx = I.reshape(_NROWS, _RPC).astype(jnp.int32)
    rt = x.T
    for _ in range(_STEPS):
        p = _stage_a(idx, rt)
        rtf = _stage_b(p.reshape(_C, _ROWS128, 128),
                       rt.reshape(_ROWS128, 128))
        rt = rtf.reshape(_G, _B)
    return _tr(rt)


# speedup vs baseline: 1.1618x; 1.1618x over previous
import jax
import jax.numpy as jnp
from jax import lax
from jax.experimental import pallas as pl
from jax.experimental.pallas import tpu as pltpu
from jax.experimental.pallas import tpu_sc as plsc

_C, _G, _S, _L = 16, 4096, 8, 2
_B = 32
_STEPS = 3
_GAMMA = 0.01
_IG = 100.0
_IG2 = 144.26950408889634
_LN2 = 0.6931471805599453
_C1 = _GAMMA * _LN2
_C2 = 2.0 * _GAMMA

_NC, _NS = 2, 16
_NW = _NC * _NS
_SLOTS = _C * _G
_SPW = _SLOTS // _NW
_CS = 8
_RPC = _CS * _S * _L
_CPW = _SPW // _CS
_OSL = 256
_CPO = _OSL // _CS
_OBW = _SPW // _OSL
_NROWS = _SLOTS * _S * _L // _RPC


def _p1(f, xs, *cs):
    return [f(x, *cs) for x in xs]


def _p2(f, xs, ys):
    return [f(x, y) for x, y in zip(xs, ys)]


def _ptree(f, pairs_list):
    while len(pairs_list) > 1:
        nxt = [_p2(f, pairs_list[i], pairs_list[i + 1])
               for i in range(0, len(pairs_list) - 1, 2)]
        if len(pairs_list) % 2:
            nxt.append(pairs_list[-1])
        pairs_list = nxt
    return pairs_list[0]


def _gamma_log_pair(accs):
    bits = _p1(lambda a: lax.bitcast_convert_type(a, jnp.int32), accs)
    e = _p1(lambda b: ((b >> 23) & 0xFF) - 127, bits)
    man = _p1(lambda b: lax.bitcast_convert_type(
        (b & 0x007FFFFF) | 0x3F800000, jnp.float32), bits)
    t = _p1(lambda mn: mn - 1.0, man)
    s = _p2(lambda tt, d: tt / d, t, _p1(lambda tt: tt + 2.0, t))
    s2 = _p2(lambda a, b: a * b, s, s)
    p = _p1(lambda q: jnp.float32(0.2) + q * jnp.float32(1.0 / 7.0), s2)
    p = _p2(lambda q, pp: jnp.float32(1.0 / 3.0) + q * pp, s2, p)
    p = _p2(lambda q, pp: 1.0 + q * pp, s2, p)
    ef = _p1(lambda ee: ee.astype(jnp.float32) * jnp.float32(_C1), e)
    sp = _p2(lambda a, b: jnp.float32(_C2) * a * b, s, p)
    return _p2(lambda a, b: a + b, ef, sp)


def _compute_chunk(rows_v, out_v, col):
    for k in range(_CS):
        base = k * 16
        rs = [[rows_v[base + j, pl.ds(lo, 16)] for lo in (0, 16)]
              for j in range(16)]
        bs = [_p2(lambda a, b: a * b, rs[2 * s], rs[2 * s + 1])
              for s in range(_S)]
        m = _ptree(jnp.maximum, bs)
        es = [_p2(lambda b, mm: jnp.exp((b - mm) * _IG), b, m) for b in bs]
        acc = _ptree(lambda a, b: a + b, es)
        gln = _gamma_log_pair(acc)
        lse = _p2(lambda mm, l_: mm + l_, m, gln)
        out_v[pl.ds((col + k) * _B, 16)] = lse[0]
        out_v[pl.ds((col + k) * _B + 16, 16)] = lse[1]


def _stage_a_body(idx_hbm, xt_hbm, p_hbm,
                  idx_v, rows_a, rows_b, out_v, sem_a, sem_b):
    cid = lax.axis_index("c")
    sid = lax.axis_index("s")
    w = sid * _NC + cid
    cc = w // 2
    gb = (w % 2) * (_G // 2)

    pltpu.sync_copy(idx_hbm.at[pl.ds(w * _CPW, _CPW), :], idx_v)

    def issue(ch, rows, sem):
        pltpu.async_copy(xt_hbm.at[idx_v.at[ch]], rows, sem)

    def wait(rows, sem):
        pltpu.make_async_copy(xt_hbm.at[idx_v.at[0]], rows, sem).wait()

    def ob_body(ob, carry):
        c0 = ob * _CPO
        issue(c0, rows_a, sem_a)

        def pair_body(p, c_):
            j0 = c0 + p * 2
            issue(j0 + 1, rows_b, sem_b)
            wait(rows_a, sem_a)
            _compute_chunk(rows_a, out_v, (p * 2) * _CS)

            @pl.when(p < _CPO // 2 - 1)
            def _():
                issue(j0 + 2, rows_a, sem_a)

            wait(rows_b, sem_b)
            _compute_chunk(rows_b, out_v, (p * 2 + 1) * _CS)
            return c_

        lax.fori_loop(0, _CPO // 2, pair_body, 0)
        pltpu.sync_copy(
            out_v,
            p_hbm.at[pl.ds(((cc * _G + gb) + ob * _OSL) * _B, _OSL * _B)])
        return carry

    lax.fori_loop(0, _OBW, ob_body, 0)


_stage_a = pl.kernel(
    _stage_a_body,
    out_type=jax.ShapeDtypeStruct((_C * _G * _B,), jnp.float32),
    mesh=plsc.VectorSubcoreMesh(core_axis_name="c", subcore_axis_name="s"),
    compiler_params=pltpu.CompilerParams(use_tc_tiling_on_sc=False),
    scratch_types=(
        pltpu.VMEM((_CPW, _RPC), jnp.int32),
        pltpu.VMEM((_RPC, _B), jnp.float32),
        pltpu.VMEM((_RPC, _B), jnp.float32),
        pltpu.VMEM((_OSL * _B,), jnp.float32),
        pltpu.SemaphoreType.DMA,
        pltpu.SemaphoreType.DMA,
    ),
)


def _stage_b_body(p_ref, rt_ref, rnt_ref):
    m1 = jnp.max(p_ref[...])
    s1 = jnp.where(m1 > 1.0, 1.0 / m1, 1.0)
    cv = p_ref[...] * s1
    mxc = jnp.max(cv, axis=0)
    acc = jnp.sum(jnp.exp((cv - mxc[None, :, :]) * _IG), axis=0)
    lse_c = mxc + _GAMMA * jnp.log(acc)
    m2 = jnp.max(lse_c)
    rr = lse_c * jnp.where(m2 > 1.0, 1.0 / m2, 1.0)
    rc = rt_ref[...]
    mx2 = jnp.maximum(rc, rr)
    z = mx2 + _GAMMA * jnp.log(jnp.exp((rc - mx2) * _IG)
                               + jnp.exp((rr - mx2) * _IG))
    m3 = jnp.max(z)
    rnt_ref[...] = z * jnp.where(m3 > 1.0, 1.0 / m3, 1.0)


_GB = _G * _B
_ROWS128 = _GB // 128

_stage_b = pl.pallas_call(
    _stage_b_body,
    out_shape=jax.ShapeDtypeStruct((_ROWS128, 128), jnp.float32),
)


def _tr_body(rt_ref, r_ref):
    r_ref[...] = rt_ref[...].T


_tr = pl.pallas_call(
    _tr_body,
    out_shape=jax.ShapeDtypeStruct((_B, _G), jnp.float32),
)


def kernel(x, I):
    idx = I.reshape(_NROWS, _RPC).astype(jnp.int32)
    rt = x.T
    for _ in range(_STEPS):
        p = _stage_a(idx, rt)
        rtf = _stage_b(p.reshape(_C, _ROWS128, 128),
                       rt.reshape(_ROWS128, 128))
        rt = rtf.reshape(_G, _B)
    return _tr(rt)


# speedup vs baseline: 1.9834x; 1.7071x over previous
import jax
import jax.numpy as jnp
from jax import lax
from jax.experimental import pallas as pl
from jax.experimental.pallas import tpu as pltpu
from jax.experimental.pallas import tpu_sc as plsc

_C, _G, _S, _L = 16, 4096, 8, 2
_B = 32
_STEPS = 3
_GAMMA = 0.01
_IG = 100.0
_IG2 = 144.26950408889634
_LN2 = 0.6931471805599453
_C1 = _GAMMA * _LN2
_C2 = 2.0 * _GAMMA

_NC, _NS = 2, 16
_NW = _NC * _NS
_SLOTS = _C * _G
_SPW = _SLOTS // _NW
_CS = 8
_RPC = _CS * _S * _L
_CPW = _SPW // _CS
_OSL = 256
_CPO = _OSL // _CS
_OBW = _SPW // _OSL
_NROWS = _SLOTS * _S * _L // _RPC


def _p1(f, xs, *cs):
    return [f(x, *cs) for x in xs]


def _p2(f, xs, ys):
    return [f(x, y) for x, y in zip(xs, ys)]


def _ptree(f, pairs_list):
    while len(pairs_list) > 1:
        nxt = [_p2(f, pairs_list[i], pairs_list[i + 1])
               for i in range(0, len(pairs_list) - 1, 2)]
        if len(pairs_list) % 2:
            nxt.append(pairs_list[-1])
        pairs_list = nxt
    return pairs_list[0]


def _compute_chunk(rows_v, outm_v, outa_v, col):
    for k2 in range(0, _CS, 2):
        g4 = [(k2, 0), (k2, 16), (k2 + 1, 0), (k2 + 1, 16)]
        bs = []
        for s in range(_S):
            r0 = [rows_v[k * 16 + 2 * s, pl.ds(lo, 16)] for k, lo in g4]
            r1 = [rows_v[k * 16 + 2 * s + 1, pl.ds(lo, 16)] for k, lo in g4]
            bs.append(_p2(lambda a, b: a * b, r0, r1))
        m = _ptree(jnp.maximum, bs)
        acc = _p2(lambda b, mm: jnp.exp((b - mm) * _IG), bs[0], m)
        for s in range(1, _S):
            e = _p2(lambda b, mm: jnp.exp((b - mm) * _IG), bs[s], m)
            acc = _p2(lambda a, b: a + b, acc, e)
        for i, (k, lo) in enumerate(g4):
            outm_v[pl.ds((col + k) * _B + lo, 16)] = m[i]
            outa_v[pl.ds((col + k) * _B + lo, 16)] = acc[i]


def _stage_a_body(idx_hbm, xt_hbm, pm_hbm, pa_hbm,
                  idx_v, rows_a, rows_b, outm_v, outa_v, sem_a, sem_b):
    cid = lax.axis_index("c")
    sid = lax.axis_index("s")
    w = sid * _NC + cid
    cc = w // 2
    gb = (w % 2) * (_G // 2)

    pltpu.sync_copy(idx_hbm.at[pl.ds(w * _CPW, _CPW), :], idx_v)

    def issue(ch, rows, sem):
        pltpu.async_copy(xt_hbm.at[idx_v.at[ch]], rows, sem)

    def wait(rows, sem):
        pltpu.make_async_copy(xt_hbm.at[idx_v.at[0]], rows, sem).wait()

    def ob_body(ob, carry):
        c0 = ob * _CPO
        issue(c0, rows_a, sem_a)

        def pair_body(p, c_):
            j0 = c0 + p * 2
            issue(j0 + 1, rows_b, sem_b)
            wait(rows_a, sem_a)
            _compute_chunk(rows_a, outm_v, outa_v, (p * 2) * _CS)

            @pl.when(p < _CPO // 2 - 1)
            def _():
                issue(j0 + 2, rows_a, sem_a)

            wait(rows_b, sem_b)
            _compute_chunk(rows_b, outm_v, outa_v, (p * 2 + 1) * _CS)
            return c_

        lax.fori_loop(0, _CPO // 2, pair_body, 0)
        off = ((cc * _G + gb) + ob * _OSL) * _B
        pltpu.sync_copy(outm_v, pm_hbm.at[pl.ds(off, _OSL * _B)])
        pltpu.sync_copy(outa_v, pa_hbm.at[pl.ds(off, _OSL * _B)])
        return carry

    lax.fori_loop(0, _OBW, ob_body, 0)


_stage_a = pl.kernel(
    _stage_a_body,
    out_type=(jax.ShapeDtypeStruct((_C * _G * _B,), jnp.float32),
              jax.ShapeDtypeStruct((_C * _G * _B,), jnp.float32)),
    mesh=plsc.VectorSubcoreMesh(core_axis_name="c", subcore_axis_name="s"),
    compiler_params=pltpu.CompilerParams(use_tc_tiling_on_sc=False),
    scratch_types=(
        pltpu.VMEM((_CPW, _RPC), jnp.int32),
        pltpu.VMEM((_RPC, _B), jnp.float32),
        pltpu.VMEM((_RPC, _B), jnp.float32),
        pltpu.VMEM((_OSL * _B,), jnp.float32),
        pltpu.VMEM((_OSL * _B,), jnp.float32),
        pltpu.SemaphoreType.DMA,
        pltpu.SemaphoreType.DMA,
    ),
)


def _stage_b_body(pm_ref, pa_ref, rt_ref, rnt_ref):
    lse_s = pm_ref[...] + _GAMMA * jnp.log(pa_ref[...])
    m1 = jnp.max(lse_s)
    s1 = jnp.where(m1 > 1.0, 1.0 / m1, 1.0)
    cv = lse_s * s1
    mxc = jnp.max(cv, axis=0)
    acc = jnp.sum(jnp.exp((cv - mxc[None, :, :]) * _IG), axis=0)
    lse_c = mxc + _GAMMA * jnp.log(acc)
    m2 = jnp.max(lse_c)
    rr = lse_c * jnp.where(m2 > 1.0, 1.0 / m2, 1.0)
    rc = rt_ref[...]
    mx2 = jnp.maximum(rc, rr)
    z = mx2 + _GAMMA * jnp.log(jnp.exp((rc - mx2) * _IG)
                               + jnp.exp((rr - mx2) * _IG))
    m3 = jnp.max(z)
    rnt_ref[...] = z * jnp.where(m3 > 1.0, 1.0 / m3, 1.0)


_GB = _G * _B
_ROWS128 = _GB // 128

_stage_b = pl.pallas_call(
    _stage_b_body,
    out_shape=jax.ShapeDtypeStruct((_ROWS128, 128), jnp.float32),
)


def _tr_body(rt_ref, r_ref):
    r_ref[...] = rt_ref[...].T


_tr = pl.pallas_call(
    _tr_body,
    out_shape=jax.ShapeDtypeStruct((_B, _G), jnp.float32),
)


def kernel(x, I):
    idx = I.reshape(_NROWS, _RPC).astype(jnp.int32)
    rt = x.T
    for _ in range(_STEPS):
        pm, pa = _stage_a(idx, rt)
        rtf = _stage_b(pm.reshape(_C, _ROWS128, 128),
                       pa.reshape(_C, _ROWS128, 128),
                       rt.reshape(_ROWS128, 128))
        rt = rtf.reshape(_G, _B)
    return _tr(rt)
